# Initial kernel scaffold; baseline (speedup 1.0000x reference)
import functools

import jax
import jax.numpy as jnp
from jax import lax
from jax.experimental import pallas as pl
from jax.experimental.pallas import tpu as pltpu
from jax.experimental.pallas import tpu_sc as plsc

N = 10000
E = 320000
D = 128

NW = 32
EPW = E // NW
CH = 80
NCH = EPW // CH

NP = 10240
HR = NP // 16

ROWS_PER_TILE = N // 16
RPC = 125

_MESH = plsc.VectorSubcoreMesh(core_axis_name="c", subcore_axis_name="s")


@functools.partial(
    pl.kernel,
    out_type=jax.ShapeDtypeStruct((2, HR, 16), jnp.float32),
    mesh=_MESH,
    scratch_types=[
        pltpu.VMEM((EPW,), jnp.int32),
        pltpu.VMEM((HR, 16), jnp.float32),
        pltpu.VMEM((HR,), jnp.int32),
        pltpu.VMEM_SHARED((HR, 16), jnp.float32),
    ],
)
def _deg_kernel(dst_hbm, out_hbm, didx_v, hist_v, iota_v, deg_sh):
    cid = lax.axis_index("c")
    sid = lax.axis_index("s")
    wid = sid * 2 + cid

    zeros16 = jnp.zeros((16,), jnp.float32)
    ones16 = jnp.ones((16,), jnp.float32)

    def _zero(i, _):
        hist_v[i, :] = zeros16
        return 0

    lax.fori_loop(0, HR, _zero, 0)

    def _iota(k, _):
        iota_v[pl.ds(k * 16, 16)] = k * 16 + lax.iota(jnp.int32, 16)
        return 0

    lax.fori_loop(0, HR // 16, _iota, 0)

    @pl.when(sid == 0)
    def _():
        pltpu.sync_copy(hist_v, deg_sh)

    plsc.subcore_barrier()

    pltpu.sync_copy(dst_hbm.at[wid], didx_v)

    def _hist(i, _):
        idx = didx_v[pl.ds(i * 16, 16)]
        plsc.addupdate_scatter(hist_v, [idx >> 4, idx & 15], ones16)
        return 0

    lax.fori_loop(0, EPW // 16, _hist, 0)

    pltpu.sync_copy(hist_v, deg_sh.at[iota_v], add=True)
    plsc.subcore_barrier()

    @pl.when(sid == 0)
    def _():
        pltpu.sync_copy(deg_sh, hist_v)
        pltpu.sync_copy(hist_v, out_hbm.at[cid])


@functools.partial(
    pl.kernel,
    out_type=jax.ShapeDtypeStruct((2, N, D), jnp.float32),
    mesh=_MESH,
    scratch_types=[
        pltpu.VMEM((NCH, CH), jnp.int32),
        pltpu.VMEM((NCH, CH), jnp.int32),
        pltpu.VMEM((CH, D), jnp.float32),
        pltpu.VMEM((RPC, D), jnp.float32),
        pltpu.VMEM_SHARED((N, D), jnp.float32),
    ],
)
def _agg_kernel(y_hbm, sidx_hbm, didx_hbm, out_hbm,
                sidx_v, didx_v, rows_v, buf_v, agg_sh):
    cid = lax.axis_index("c")
    sid = lax.axis_index("s")
    wid = sid * 2 + cid

    zeros16 = jnp.zeros((16,), jnp.float32)

    def _zrow(i, _):
        def _zcol(j, _):
            buf_v[i, pl.ds(j * 16, 16)] = zeros16
            return 0

        lax.fori_loop(0, D // 16, _zcol, 0)
        return 0

    lax.fori_loop(0, RPC, _zrow, 0)

    base = sid * ROWS_PER_TILE

    def _zs(k, _):
        pltpu.sync_copy(buf_v, agg_sh.at[pl.ds(base + k * RPC, RPC)])
        return 0

    lax.fori_loop(0, ROWS_PER_TILE // RPC, _zs, 0)
    plsc.subcore_barrier()

    pltpu.sync_copy(sidx_hbm.at[wid], sidx_v)
    pltpu.sync_copy(didx_hbm.at[wid], didx_v)

    def _edge_chunk(c, _):
        pltpu.sync_copy(y_hbm.at[sidx_v.at[c]], rows_v)
        pltpu.sync_copy(rows_v, agg_sh.at[didx_v.at[c]], add=True)
        return 0

    lax.fori_loop(0, NCH, _edge_chunk, 0)
    plsc.subcore_barrier()

    def _dump(k, _):
        r0 = base + k * RPC
        pltpu.sync_copy(agg_sh.at[pl.ds(r0, RPC)], buf_v)
        pltpu.sync_copy(buf_v, out_hbm.at[cid, pl.ds(r0, RPC)])
        return 0

    lax.fori_loop(0, ROWS_PER_TILE // RPC, _dump, 0)


R = 1000


def _y_body(h_ref, m_ref, w_ref, y_ref, dinv_ref):
    deg = h_ref[0] + h_ref[1] + 1.0
    dinv = lax.rsqrt(deg)
    dinv_ref[...] = dinv
    y_ref[...] = jnp.dot(m_ref[...], w_ref[...],
                         preferred_element_type=jnp.float32) * dinv


_y_call = pl.pallas_call(
    _y_body,
    grid=(N // R,),
    in_specs=[
        pl.BlockSpec((2, R, 1), lambda i: (0, i, 0)),
        pl.BlockSpec((R, D), lambda i: (i, 0)),
        pl.BlockSpec((D, D), lambda i: (0, 0)),
    ],
    out_specs=[
        pl.BlockSpec((R, D), lambda i: (i, 0)),
        pl.BlockSpec((R, 1), lambda i: (i, 0)),
    ],
    out_shape=[
        jax.ShapeDtypeStruct((N, D), jnp.float32),
        jax.ShapeDtypeStruct((N, 1), jnp.float32),
    ],
)


def _final_body(p_ref, y_ref, dinv_ref, b_ref, o_ref):
    acc = p_ref[0] + p_ref[1] + y_ref[...]
    o_ref[...] = jnp.maximum(dinv_ref[...] * acc + b_ref[...], 0.0)


_final_call = pl.pallas_call(
    _final_body,
    grid=(N // R,),
    in_specs=[
        pl.BlockSpec((2, R, D), lambda i: (0, i, 0)),
        pl.BlockSpec((R, D), lambda i: (i, 0)),
        pl.BlockSpec((R, 1), lambda i: (i, 0)),
        pl.BlockSpec((1, D), lambda i: (0, 0)),
    ],
    out_specs=pl.BlockSpec((R, D), lambda i: (i, 0)),
    out_shape=jax.ShapeDtypeStruct((N, D), jnp.float32),
)


def kernel(map_tensor, edge_index, W, b):
    ei = edge_index.astype(jnp.int32)
    src3 = ei[0].reshape(NW, NCH, CH)
    dst2 = ei[1].reshape(NW, EPW)
    dst3 = ei[1].reshape(NW, NCH, CH)

    hist = _deg_kernel(dst2)
    deg2 = hist.reshape(2, NP)[:, :N].reshape(2, N, 1)
    y, dinv = _y_call(deg2, map_tensor, W)
    p = _agg_kernel(y, src3, dst3)
    return _final_call(p, y, dinv, b.reshape(1, D))


# speedup vs baseline: 6.3872x; 6.3872x over previous
import functools

import jax
import jax.numpy as jnp
from jax import lax
from jax.experimental import pallas as pl
from jax.experimental.pallas import tpu as pltpu
from jax.experimental.pallas import tpu_sc as plsc

N = 10000
E = 320000
D = 128

NW = 32
EPW = E // NW
NRD = 79
EPD = NRD * 128

EPT = E // 16
NRA = 157
EPA = NRA * 128

NP = 10240
SEG = NP // 16

HALF = NP // 2
AGG_ROWS = 5248
SEGA = AGG_ROWS // 16

PAD_DST = 16383

_MESH = plsc.VectorSubcoreMesh(core_axis_name="c", subcore_axis_name="s")


@functools.partial(
    pl.kernel,
    out_type=jax.ShapeDtypeStruct((2, NP, 16), jnp.float32),
    mesh=_MESH,
    scratch_types=[
        pltpu.VMEM((NRD, 128), jnp.int32),
        pltpu.VMEM((16, 16), jnp.float32),
        pltpu.VMEM_SHARED((NP, 16), jnp.float32),
    ],
)
def _deg_kernel(pk_hbm, z16_hbm, out_hbm, didx_v, ones_v, hist_sh):
    cid = lax.axis_index("c")
    sid = lax.axis_index("s")
    wid = sid * 2 + cid

    ones16 = jnp.ones((16,), jnp.float32)

    def _fill_ones(i, _):
        ones_v[i, :] = ones16
        return 0

    lax.fori_loop(0, 16, _fill_ones, 0)

    seg0 = sid * SEG
    pltpu.sync_copy(z16_hbm, hist_sh.at[pl.ds(seg0, SEG)])

    pltpu.sync_copy(pk_hbm.at[wid], didx_v)

    def _unpack(i, _):
        def _u16(k, _):
            sl = pl.ds(k * 16, 16)
            didx_v[i, sl] = lax.shift_right_logical(didx_v[i, sl], 14)
            return 0

        lax.fori_loop(0, 8, _u16, 0)
        return 0

    lax.fori_loop(0, NRD, _unpack, 0)
    plsc.subcore_barrier()

    def _chunk(c, _):
        def _sub(k, _):
            pltpu.sync_copy(
                ones_v, hist_sh.at[didx_v.at[c, pl.ds(k * 16, 16)]],
                add=True)
            return 0

        lax.fori_loop(0, 8, _sub, 0)
        return 0

    lax.fori_loop(0, NRD, _chunk, 0)
    plsc.subcore_barrier()

    pltpu.sync_copy(hist_sh.at[pl.ds(seg0, SEG)],
                    out_hbm.at[cid, pl.ds(seg0, SEG)])


@functools.partial(
    pl.kernel,
    out_type=jax.ShapeDtypeStruct((2, HALF, D), jnp.float32),
    mesh=_MESH,
    scratch_types=[
        pltpu.VMEM((NRA, 128), jnp.int32),
        pltpu.VMEM((NRA, 128), jnp.int32),
        pltpu.VMEM((16, D), jnp.float32),
        pltpu.VMEM_SHARED((AGG_ROWS, D), jnp.float32),
    ],
)
def _agg_kernel(y_hbm, pk_hbm, z128_hbm, out_hbm,
                didx_v, sidx_v, rows_v, agg_sh):
    cid = lax.axis_index("c")
    sid = lax.axis_index("s")

    pltpu.sync_copy(pk_hbm.at[sid], didx_v.at[pl.ds(0, NRA)])

    lo = cid * HALF

    def _unpack(i, _):
        def _u16(k, _):
            sl = pl.ds(k * 16, 16)
            v = didx_v[i, sl]
            sidx_v[i, sl] = v & 16383
            local = lax.shift_right_logical(v, 14) - lo
            ok = (local >= 0) & (local < HALF)
            didx_v[i, sl] = jnp.where(ok, local, HALF)
            return 0

        lax.fori_loop(0, 8, _u16, 0)
        return 0

    lax.fori_loop(0, NRA, _unpack, 0)

    sega0 = sid * SEGA

    def _zs(k, _):
        pltpu.sync_copy(z128_hbm, agg_sh.at[pl.ds(sega0 + k * 64, 64)])
        return 0

    lax.fori_loop(0, SEGA // 64, _zs, 0)
    pltpu.sync_copy(z128_hbm.at[pl.ds(0, 8)],
                    agg_sh.at[pl.ds(sega0 + 320, 8)])
    plsc.subcore_barrier()

    def _edge_chunk(c, _):
        def _sub(k, _):
            sl = pl.ds(k * 16, 16)
            pltpu.sync_copy(y_hbm.at[sidx_v.at[c, sl]], rows_v)
            pltpu.sync_copy(rows_v, agg_sh.at[didx_v.at[c, sl]],
                            add=True)
            return 0

        lax.fori_loop(0, 8, _sub, 0)
        return 0

    lax.fori_loop(0, NRA, _edge_chunk, 0)
    plsc.subcore_barrier()

    @pl.when(sid < 15)
    def _():
        pltpu.sync_copy(agg_sh.at[pl.ds(sega0, SEGA)],
                        out_hbm.at[cid, pl.ds(sega0, SEGA)])

    @pl.when(sid == 15)
    def _():
        n_left = HALF - 15 * SEGA
        pltpu.sync_copy(agg_sh.at[pl.ds(15 * SEGA, n_left)],
                        out_hbm.at[cid, pl.ds(15 * SEGA, n_left)])


RY = 1000


def _y_body(h_ref, m_ref, w_ref, xw_ref, y_ref, dinv_ref):
    deg = h_ref[0, :, :1] + h_ref[1, :, :1] + 1.0
    dinv = lax.rsqrt(deg)
    dinv_ref[...] = dinv
    xw = jnp.dot(m_ref[...], w_ref[...], preferred_element_type=jnp.float32)
    xw_ref[...] = xw
    y_ref[...] = xw * dinv


_y_call = pl.pallas_call(
    _y_body,
    grid=(N // RY,),
    in_specs=[
        pl.BlockSpec((2, RY, 16), lambda i: (0, i, 0)),
        pl.BlockSpec((RY, D), lambda i: (i, 0)),
        pl.BlockSpec((D, D), lambda i: (0, 0)),
    ],
    out_specs=[
        pl.BlockSpec((RY, D), lambda i: (i, 0)),
        pl.BlockSpec((RY, D), lambda i: (i, 0)),
        pl.BlockSpec((RY, 1), lambda i: (i, 0)),
    ],
    out_shape=[
        jax.ShapeDtypeStruct((N, D), jnp.float32),
        jax.ShapeDtypeStruct((N, D), jnp.float32),
        jax.ShapeDtypeStruct((N, 1), jnp.float32),
    ],
)


RF = 512
PB = HALF // RF


def _final_body(p_ref, xw_ref, dinv_ref, b_ref, o_ref):
    dinv = dinv_ref[...]
    o_ref[...] = jnp.maximum(
        dinv * p_ref[0] + dinv * dinv * xw_ref[...] + b_ref[...], 0.0)


_final_call = pl.pallas_call(
    _final_body,
    grid=(NP // RF,),
    in_specs=[
        pl.BlockSpec((1, RF, D), lambda i: (i // PB, i % PB, 0)),
        pl.BlockSpec((RF, D), lambda i: (i, 0)),
        pl.BlockSpec((RF, 1), lambda i: (i, 0)),
        pl.BlockSpec((1, D), lambda i: (0, 0)),
    ],
    out_specs=pl.BlockSpec((RF, D), lambda i: (i, 0)),
    out_shape=jax.ShapeDtypeStruct((NP, D), jnp.float32),
)


def kernel(map_tensor, edge_index, W, b):
    ei = edge_index.astype(jnp.int32)
    packed = ei[0] | (ei[1] << 14)
    padval = jnp.int32(PAD_DST << 14)
    pk_deg = jnp.pad(packed.reshape(NW, EPW), ((0, 0), (0, EPD - EPW)),
                     constant_values=padval).reshape(NW, NRD, 128)
    pk_agg = jnp.pad(packed.reshape(16, EPT), ((0, 0), (0, EPA - EPT)),
                     constant_values=padval).reshape(16, NRA, 128)
    z16 = jnp.zeros((SEG, 16), jnp.float32)
    z128 = jnp.zeros((64, D), jnp.float32)

    hist = _deg_kernel(pk_deg, z16)
    xw, y, dinv = _y_call(hist, map_tensor, W)
    p = _agg_kernel(y, pk_agg, z128)
    return _final_call(p, xw, dinv, b.reshape(1, D))[:N]
